# baseline (speedup 1.0000x reference)
import functools

import jax
import jax.numpy as jnp
from jax.experimental import pallas as pl
from jax.experimental.pallas import tpu as pltpu


def _pick_block(n, pref):
    return pref if n % pref == 0 else n


def _prologue_body(x_ref, w_ref, asd_ref,
                   hb_ref, pp_ref, pp2_ref, qq_ref):
    w = w_ref[...]
    wasd = jnp.dot(w, asd_ref[...], preferred_element_type=jnp.float32)
    hx = jnp.dot(x_ref[...], jnp.concatenate([w, wasd], axis=1),
                 preferred_element_type=jnp.float32)
    d = w.shape[1]
    h = hx[:, :d]
    aa = hx[:, d:]
    hb_ref[:, :d] = h.astype(jnp.bfloat16)
    hb_ref[:, d:] = jnp.ones((h.shape[0], 8), jnp.bfloat16)
    a_s = aa[:, :8]
    a_d = aa[:, 8:]
    s_max = jnp.max(a_s)
    v = a_d + s_max
    m = jnp.maximum(v, 0.2 * v)
    pp_ref[...] = jnp.exp(a_d - m).astype(jnp.bfloat16)
    pp2_ref[...] = jnp.exp(0.2 * a_d - m).astype(jnp.bfloat16)
    a_sT = jnp.transpose(aa[:, 0:1])
    qq_ref[0:4, :] = jnp.broadcast_to(jnp.exp(a_sT), (4, a_sT.shape[1])
                                      ).astype(jnp.bfloat16)
    qq_ref[4:8, :] = jnp.broadcast_to(jnp.exp(0.2 * a_sT), (4, a_sT.shape[1])
                                      ).astype(jnp.bfloat16)


def _prologue(x, W, a_src, a_dst):
    n, d = x.shape
    asd = jnp.concatenate([jnp.broadcast_to(a_src[:, None], (d, 8)),
                           jnp.broadcast_to(a_dst[:, None], (d, 8))], axis=1)
    out_shapes = (
        jax.ShapeDtypeStruct((n, d + 8), jnp.bfloat16),
        jax.ShapeDtypeStruct((n, 8), jnp.bfloat16),
        jax.ShapeDtypeStruct((n, 8), jnp.bfloat16),
        jax.ShapeDtypeStruct((8, n), jnp.bfloat16),
    )
    return pl.pallas_call(
        _prologue_body,
        out_shape=out_shapes,
    )(x, W, asd)


def _main_body(adj_ref, qq_ref, pp_ref, pp2_ref, hb_ref, b_ref,
               out_ref, *rest, apply_elu, emit_mask):
    a = adj_ref[...]
    abf = a.astype(jnp.bfloat16)
    if emit_mask:
        rest[0][...] = a.astype(jnp.int2)
    qb = qq_ref[0:1, :]
    q2b = qq_ref[4:5, :]
    ppb = pp_ref[:, 0:1]
    pp2b = pp2_ref[:, 0:1]
    t = jnp.maximum(ppb * qb, pp2b * q2b)
    w = t * abf

    numden = jax.lax.dot_general(
        w, hb_ref[...],
        (((1,), (0,)), ((), ())), preferred_element_type=jnp.float32)
    d = out_ref.shape[1]
    num = numden[:, :d]
    den = numden[:, d:d + 1]
    out = num / (den + 1e-16) + b_ref[0:1, :]
    if apply_elu:
        out = jnp.where(out > 0, out, jnp.exp(out) - 1.0)
    out_ref[...] = out


def _gat_layer(adj, x, W, a_src, a_dst, b, *, apply_elu, emit_mask, bi_pref):
    n, d = x.shape
    bi = _pick_block(n, bi_pref)
    hbe, pp, pp2, qq = _prologue(x, W, a_src, a_dst)
    b_row = b.reshape(1, d)

    grid = (n // bi,)
    in_specs = [
        pl.BlockSpec((bi, n), lambda i: (i, 0)),
        pl.BlockSpec((8, n), lambda i: (0, 0)),
        pl.BlockSpec((bi, 8), lambda i: (i, 0)),
        pl.BlockSpec((bi, 8), lambda i: (i, 0)),
        pl.BlockSpec((n, d + 8), lambda i: (0, 0)),
        pl.BlockSpec((1, d), lambda i: (0, 0)),
    ]
    out_shapes = [jax.ShapeDtypeStruct((n, d), jnp.float32)]
    out_specs = [pl.BlockSpec((bi, d), lambda i: (i, 0))]
    if emit_mask:
        out_shapes.append(jax.ShapeDtypeStruct((n, n), jnp.int2))
        out_specs.append(pl.BlockSpec((bi, n), lambda i: (i, 0)))
    body = functools.partial(_main_body, apply_elu=apply_elu,
                             emit_mask=emit_mask)
    outs = pl.pallas_call(
        body,
        grid=grid,
        in_specs=in_specs,
        out_specs=out_specs,
        out_shape=out_shapes,
        compiler_params=pltpu.CompilerParams(
            dimension_semantics=("arbitrary",),
        ),
    )(adj, qq, pp, pp2, hbe, b_row)
    if emit_mask:
        return outs[0], outs[1]
    return outs[0], None


def kernel(adj, x, W1, a_src1, a_dst1, b1, W2, a_src2, a_dst2, b2):
    h1, mask8 = _gat_layer(adj, x, W1, a_src1, a_dst1, b1,
                           apply_elu=True, emit_mask=True, bi_pref=400)
    out, _ = _gat_layer(mask8, h1, W2, a_src2, a_dst2, b2,
                        apply_elu=False, emit_mask=False, bi_pref=400)
    return out


# speedup vs baseline: 1.0670x; 1.0670x over previous
import functools

import jax
import jax.numpy as jnp
from jax.experimental import pallas as pl
from jax.experimental.pallas import tpu as pltpu


def _pick_block(n, pref):
    return pref if n % pref == 0 else n


def _layer_body(adj_ref, x_ref, w_ref, asd_ref, b_ref,
                out_ref, *rest, bi, apply_elu, emit_mask):
    if emit_mask:
        mask_ref, hb_s, pp_s, pp2_s, qq_s = rest
    else:
        hb_s, pp_s, pp2_s, qq_s = rest
    i = pl.program_id(0)

    @pl.when(i == 0)
    def _prologue():
        w = w_ref[...]
        wasd = jnp.dot(w, asd_ref[...], preferred_element_type=jnp.float32)
        hx = jnp.dot(x_ref[...], jnp.concatenate([w, wasd], axis=1),
                     preferred_element_type=jnp.float32)
        d = w.shape[1]
        h = hx[:, :d]
        aa = hx[:, d:]
        hb_s[:, :d] = h.astype(jnp.bfloat16)
        hb_s[:, d:] = jnp.ones((h.shape[0], 8), jnp.bfloat16)
        a_s = aa[:, :8]
        a_d = aa[:, 8:]
        s_max = jnp.max(a_s)
        v = a_d + s_max
        m = jnp.maximum(v, 0.2 * v)
        pp_s[...] = jnp.exp(a_d - m).astype(jnp.bfloat16)
        pp2_s[...] = jnp.exp(0.2 * a_d - m).astype(jnp.bfloat16)
        a_sT = jnp.transpose(aa[:, 0:1])
        both = jnp.concatenate([a_sT, 0.2 * a_sT], axis=0)
        qq_s[...] = jnp.exp(both).astype(jnp.bfloat16)

    a = adj_ref[...]
    abf = a.astype(jnp.bfloat16)
    if emit_mask:
        mask_ref[...] = a.astype(jnp.int2)
    qb = qq_s[0:1, :]
    q2b = qq_s[1:2, :]
    ppb = pp_s[pl.ds(i * bi, bi), 0:1]
    pp2b = pp2_s[pl.ds(i * bi, bi), 0:1]
    t = jnp.maximum(ppb * qb, pp2b * q2b)
    w = t * abf

    numden = jax.lax.dot_general(
        w, hb_s[...],
        (((1,), (0,)), ((), ())), preferred_element_type=jnp.float32)
    d = out_ref.shape[1]
    num = numden[:, :d]
    den = numden[:, d:d + 1]
    out = num / (den + 1e-16) + b_ref[0:1, :]
    if apply_elu:
        out = jnp.where(out > 0, out, jnp.exp(out) - 1.0)
    out_ref[...] = out


def _gat_layer(adj, x, W, a_src, a_dst, b, *, apply_elu, emit_mask, bi_pref):
    n, d = x.shape
    bi = _pick_block(n, bi_pref)
    asd = jnp.concatenate([jnp.broadcast_to(a_src[:, None], (d, 8)),
                           jnp.broadcast_to(a_dst[:, None], (d, 8))], axis=1)
    b_row = b.reshape(1, d)

    grid = (n // bi,)
    in_specs = [
        pl.BlockSpec((bi, n), lambda i: (i, 0)),
        pl.BlockSpec((n, d), lambda i: (0, 0)),
        pl.BlockSpec((d, d), lambda i: (0, 0)),
        pl.BlockSpec((d, 16), lambda i: (0, 0)),
        pl.BlockSpec((1, d), lambda i: (0, 0)),
    ]
    out_shapes = [jax.ShapeDtypeStruct((n, d), jnp.float32)]
    out_specs = [pl.BlockSpec((bi, d), lambda i: (i, 0))]
    if emit_mask:
        out_shapes.append(jax.ShapeDtypeStruct((n, n), jnp.int2))
        out_specs.append(pl.BlockSpec((bi, n), lambda i: (i, 0)))
    body = functools.partial(_layer_body, bi=bi, apply_elu=apply_elu,
                             emit_mask=emit_mask)
    outs = pl.pallas_call(
        body,
        grid=grid,
        in_specs=in_specs,
        out_specs=out_specs,
        out_shape=out_shapes,
        scratch_shapes=[
            pltpu.VMEM((n, d + 8), jnp.bfloat16),
            pltpu.VMEM((n, 8), jnp.bfloat16),
            pltpu.VMEM((n, 8), jnp.bfloat16),
            pltpu.VMEM((2, n), jnp.bfloat16),
        ],
        compiler_params=pltpu.CompilerParams(
            dimension_semantics=("arbitrary",),
            vmem_limit_bytes=100 * 1024 * 1024,
        ),
    )(adj, x, W, asd, b_row)
    if emit_mask:
        return outs[0], outs[1]
    return outs[0], None


def kernel(adj, x, W1, a_src1, a_dst1, b1, W2, a_src2, a_dst2, b2):
    h1, mask2 = _gat_layer(adj, x, W1, a_src1, a_dst1, b1,
                           apply_elu=True, emit_mask=True, bi_pref=400)
    out, _ = _gat_layer(mask2, h1, W2, a_src2, a_dst2, b2,
                        apply_elu=False, emit_mask=False, bi_pref=400)
    return out
